# baseline (speedup 1.0000x reference)
import functools
import math

import jax
import jax.numpy as jnp
import numpy as np
from jax import lax
from jax.experimental import pallas as pl
from jax.experimental.pallas import tpu as pltpu
from jax.experimental.pallas import tpu_sc as plsc

D_MODEL = 1024
POS_MAX = 300
TYPE_VOCAB = 4
LN_EPS = 1e-12

POS_PAD = 304
CTAB = 1216
TOK_OFF = 384
TAB = 512
N_ROWS = 16384
N_SC = 1024
BLK = 1024

NC, NS = 2, 16
NW = NC * NS
ROWS_PER_W = N_SC // NW
CH = 32
NCHUNK = ROWS_PER_W // CH
LANES = 16
NVEC = D_MODEL // LANES


def _make_pe_np(d_model, max_len):
    position = np.arange(max_len, dtype=np.float32)[:, None]
    div_term = np.exp(
        np.arange(0, d_model, 2, dtype=np.float32) * -(math.log(1000.0) / d_model)
    )
    pe = np.zeros((max_len, d_model), dtype=np.float32)
    pe[:, 0::2] = np.sin(position * div_term)
    pe[:, 1::2] = np.cos(position * div_term)
    return pe


_PE_POS = np.zeros((POS_PAD, D_MODEL), dtype=np.float32)
_PE_POS[:POS_MAX] = _make_pe_np(D_MODEL, POS_MAX)
_PE_TOK = np.zeros((8, D_MODEL), dtype=np.float32)
_PE_TOK[:TYPE_VOCAB] = _make_pe_np(D_MODEL, TYPE_VOCAB)

_R_EXP = np.zeros((CTAB, POS_PAD), dtype=np.float32)
_S_EXP = np.zeros((CTAB, 8), dtype=np.float32)
for _i in range(POS_MAX * TYPE_VOCAB):
    _R_EXP[_i, _i // TYPE_VOCAB] = 1.0
    _S_EXP[_i, _i % TYPE_VOCAB] = 1.0


def _proj_body(pe_pos_ref, pe_tok_ref, wpt_ref, bp_ref, wtt_ref, bt_ref,
               r_ref, s_ref, c_ref, t_ref):
    a = (
        jnp.dot(pe_pos_ref[...], wpt_ref[...], preferred_element_type=jnp.float32)
        + bp_ref[...]
    )
    tk = (
        jnp.dot(pe_tok_ref[...], wtt_ref[...], preferred_element_type=jnp.float32)
        + bt_ref[...]
    )
    c_ref[...] = (
        jnp.dot(r_ref[...], a, preferred_element_type=jnp.float32)
        + jnp.dot(s_ref[...], tk, preferred_element_type=jnp.float32)
    )
    t_ref[0:POS_PAD, :] = a
    t_ref[POS_PAD:TOK_OFF, :] = jnp.zeros((TOK_OFF - POS_PAD, D_MODEL), jnp.float32)
    t_ref[TOK_OFF:TOK_OFF + 8, :] = tk
    t_ref[TOK_OFF + 8:TAB, :] = jnp.zeros((TAB - TOK_OFF - 8, D_MODEL), jnp.float32)


def _sc_body(x_hbm, pid_hbm, tid_hbm, c_hbm, gamma_hbm, beta_hbm, out_hbm,
             x_v, g_v, pid_v, tid_v, idx_v, gb_v, red_v, dsem):
    wid = lax.axis_index("s") * NC + lax.axis_index("c")
    base = wid * ROWS_PER_W
    pltpu.sync_copy(gamma_hbm, gb_v.at[0])
    pltpu.sync_copy(beta_hbm, gb_v.at[1])

    def chunk_body(ci, carry):
        off = base + ci * CH
        pltpu.sync_copy(pid_hbm.at[pl.ds(off, CH)], pid_v)
        pltpu.sync_copy(tid_hbm.at[pl.ds(off, CH)], tid_v)
        for j in range(CH // LANES):
            sl = pl.ds(j * LANES, LANES)
            idx_v[sl] = pid_v[sl] * TYPE_VOCAB + tid_v[sl]
        gcopy = pltpu.async_copy(c_hbm.at[idx_v], g_v, dsem)
        pltpu.sync_copy(x_hbm.at[pl.ds(off, CH)], x_v)
        gcopy.wait()

        def row_body(r, rcarry):
            zero = jnp.zeros((LANES,), jnp.float32)
            acc_s = [zero, zero, zero, zero]
            acc_q = [zero, zero, zero, zero]
            for j in range(NVEC):
                sl = pl.ds(j * LANES, LANES)
                v = x_v[r, sl] + g_v[r, sl]
                x_v[r, sl] = v
                acc_s[j % 4] = acc_s[j % 4] + v
                acc_q[j % 4] = acc_q[j % 4] + v * v
            s = (acc_s[0] + acc_s[1]) + (acc_s[2] + acc_s[3])
            q = (acc_q[0] + acc_q[1]) + (acc_q[2] + acc_q[3])
            lane = lax.iota(jnp.int32, LANES)
            for k in (8, 4, 2, 1):
                red_v[pl.ds(0, LANES)] = s
                red_v[pl.ds(LANES, LANES)] = q
                perm = lax.bitwise_xor(lane, k)
                s = s + plsc.load_gather(red_v, [perm])
                q = q + plsc.load_gather(red_v, [perm + LANES])
            m = s * (1.0 / D_MODEL)
            var = q * (1.0 / D_MODEL) - m * m
            vv = var + LN_EPS
            bits = plsc.bitcast(vv, jnp.int32)
            y = plsc.bitcast(
                jnp.full((LANES,), 0x5F3759DF, jnp.int32)
                - lax.shift_right_arithmetic(bits, 1),
                jnp.float32,
            )
            h = vv * 0.5
            for _ in range(4):
                y = y * (1.5 - h * y * y)
            for j in range(NVEC):
                sl = pl.ds(j * LANES, LANES)
                v = x_v[r, sl]
                x_v[r, sl] = (v - m) * y * gb_v[0, sl] + gb_v[1, sl]
            return rcarry

        lax.fori_loop(0, CH, row_body, 0)
        pltpu.sync_copy(x_v, out_hbm.at[pl.ds(off, CH)])
        return carry

    lax.fori_loop(0, NCHUNK, chunk_body, 0)


def _fused_body(x_ref, pid_ref, tid_ref, t_ref, g_ref, b_ref, o_ref):
    x = x_ref[...]
    pid = pid_ref[...]
    tid = tid_ref[...]
    iota = jax.lax.broadcasted_iota(jnp.int32, (BLK, TAB), 1)
    sel = ((iota == pid) | (iota == tid + TOK_OFF)).astype(jnp.float32)
    x = x + jnp.dot(sel, t_ref[...], preferred_element_type=jnp.float32)
    mean = jnp.mean(x, axis=1, keepdims=True)
    xc = x - mean
    var = jnp.mean(xc * xc, axis=1, keepdims=True)
    o_ref[...] = xc * (jax.lax.rsqrt(var + LN_EPS) * g_ref[...]) + b_ref[...]


@jax.jit
def kernel(token_type_ids, position_ids, inputs_embeds, W_pos, b_pos,
           W_tok, b_tok, ln_gamma, ln_beta):
    B, S, D = inputs_embeds.shape
    n = B * S
    x = inputs_embeds.reshape(n, D)
    pid = position_ids.reshape(n).astype(jnp.int32)
    tid = token_type_ids.reshape(n).astype(jnp.int32)

    ctab, ttab = pl.pallas_call(
        _proj_body,
        out_shape=(
            jax.ShapeDtypeStruct((CTAB, D_MODEL), jnp.float32),
            jax.ShapeDtypeStruct((TAB, D_MODEL), jnp.float32),
        ),
    )(
        jnp.asarray(_PE_POS), jnp.asarray(_PE_TOK),
        W_pos.T, b_pos.reshape(1, D_MODEL),
        W_tok.T, b_tok.reshape(1, D_MODEL),
        jnp.asarray(_R_EXP), jnp.asarray(_S_EXP),
    )

    n_tc = n - N_SC
    sc_blocks = N_SC // BLK
    out_tc = pl.pallas_call(
        _fused_body,
        grid=(n_tc // BLK,),
        in_specs=[
            pl.BlockSpec((BLK, D_MODEL), lambda i: (i + sc_blocks, 0)),
            pl.BlockSpec((BLK, 1), lambda i: (i + sc_blocks, 0)),
            pl.BlockSpec((BLK, 1), lambda i: (i + sc_blocks, 0)),
            pl.BlockSpec((TAB, D_MODEL), lambda i: (0, 0)),
            pl.BlockSpec((1, D_MODEL), lambda i: (0, 0)),
            pl.BlockSpec((1, D_MODEL), lambda i: (0, 0)),
        ],
        out_specs=pl.BlockSpec((BLK, D_MODEL), lambda i: (i, 0)),
        out_shape=jax.ShapeDtypeStruct((n_tc, D_MODEL), jnp.float32),
    )(
        x, pid.reshape(n, 1), tid.reshape(n, 1), ttab,
        ln_gamma.reshape(1, D_MODEL), ln_beta.reshape(1, D_MODEL),
    )

    mesh = plsc.VectorSubcoreMesh(core_axis_name="c", subcore_axis_name="s")
    sc = functools.partial(
        pl.kernel,
        mesh=mesh,
        compiler_params=pltpu.CompilerParams(needs_layout_passes=False),
        out_type=jax.ShapeDtypeStruct((N_SC, D), jnp.float32),
        scratch_types=[
            pltpu.VMEM((CH, D_MODEL), jnp.float32),
            pltpu.VMEM((CH, D_MODEL), jnp.float32),
            pltpu.VMEM((CH,), jnp.int32),
            pltpu.VMEM((CH,), jnp.int32),
            pltpu.VMEM((CH,), jnp.int32),
            pltpu.VMEM((2, D_MODEL), jnp.float32),
            pltpu.VMEM((2 * LANES,), jnp.float32),
            pltpu.SemaphoreType.DMA,
        ],
    )(_sc_body)
    out_sc = sc(x, pid, tid, ctab, ln_gamma, ln_beta)

    out = jnp.concatenate([out_sc, out_tc], axis=0)
    return out.reshape(B, S, D)


# speedup vs baseline: 1.0117x; 1.0117x over previous
import functools
import math

import jax
import jax.numpy as jnp
import numpy as np
from jax import lax
from jax.experimental import pallas as pl
from jax.experimental.pallas import tpu as pltpu
from jax.experimental.pallas import tpu_sc as plsc

D_MODEL = 1024
POS_MAX = 300
TYPE_VOCAB = 4
LN_EPS = 1e-12

POS_PAD = 304
CTAB = 1216
TOK_OFF = 384
TAB = 512
N_ROWS = 16384
N_SC = 1024
BLK = 1024

NC, NS = 1, 16
NW = NC * NS
ROWS_PER_W = N_SC // NW
CH = 32
NCHUNK = ROWS_PER_W // CH
LANES = 16
NVEC = D_MODEL // LANES


def _make_pe_np(d_model, max_len):
    position = np.arange(max_len, dtype=np.float32)[:, None]
    div_term = np.exp(
        np.arange(0, d_model, 2, dtype=np.float32) * -(math.log(1000.0) / d_model)
    )
    pe = np.zeros((max_len, d_model), dtype=np.float32)
    pe[:, 0::2] = np.sin(position * div_term)
    pe[:, 1::2] = np.cos(position * div_term)
    return pe


_PE_POS = np.zeros((POS_PAD, D_MODEL), dtype=np.float32)
_PE_POS[:POS_MAX] = _make_pe_np(D_MODEL, POS_MAX)
_PE_TOK = np.zeros((8, D_MODEL), dtype=np.float32)
_PE_TOK[:TYPE_VOCAB] = _make_pe_np(D_MODEL, TYPE_VOCAB)

_R_EXP = np.zeros((CTAB, POS_PAD), dtype=np.float32)
_S_EXP = np.zeros((CTAB, 8), dtype=np.float32)
for _i in range(POS_MAX * TYPE_VOCAB):
    _R_EXP[_i, _i // TYPE_VOCAB] = 1.0
    _S_EXP[_i, _i % TYPE_VOCAB] = 1.0


def _proj_body(pe_pos_ref, pe_tok_ref, wpt_ref, bp_ref, wtt_ref, bt_ref,
               r_ref, s_ref, c_ref, t_ref):
    a = (
        jnp.dot(pe_pos_ref[...], wpt_ref[...], preferred_element_type=jnp.float32)
        + bp_ref[...]
    )
    tk = (
        jnp.dot(pe_tok_ref[...], wtt_ref[...], preferred_element_type=jnp.float32)
        + bt_ref[...]
    )
    c_ref[...] = (
        jnp.dot(r_ref[...], a, preferred_element_type=jnp.float32)
        + jnp.dot(s_ref[...], tk, preferred_element_type=jnp.float32)
    )
    t_ref[0:POS_PAD, :] = a
    t_ref[POS_PAD:TOK_OFF, :] = jnp.zeros((TOK_OFF - POS_PAD, D_MODEL), jnp.float32)
    t_ref[TOK_OFF:TOK_OFF + 8, :] = tk
    t_ref[TOK_OFF + 8:TAB, :] = jnp.zeros((TAB - TOK_OFF - 8, D_MODEL), jnp.float32)


def _sc_body(x_hbm, pid_hbm, tid_hbm, c_hbm, gamma_hbm, beta_hbm, out_hbm,
             x_v, g_v, pid_v, tid_v, idx_v, gb_v, red_v, dsem):
    wid = lax.axis_index("s") * NC + lax.axis_index("c")
    base = wid * ROWS_PER_W
    pltpu.sync_copy(gamma_hbm, gb_v.at[0])
    pltpu.sync_copy(beta_hbm, gb_v.at[1])

    def chunk_body(ci, carry):
        off = base + ci * CH
        pltpu.sync_copy(pid_hbm.at[pl.ds(off, CH)], pid_v)
        pltpu.sync_copy(tid_hbm.at[pl.ds(off, CH)], tid_v)
        for j in range(CH // LANES):
            sl = pl.ds(j * LANES, LANES)
            idx_v[sl] = pid_v[sl] * TYPE_VOCAB + tid_v[sl]
        gcopy = pltpu.async_copy(c_hbm.at[idx_v], g_v, dsem)
        pltpu.sync_copy(x_hbm.at[pl.ds(off, CH)], x_v)
        gcopy.wait()

        def row_body(r, rcarry):
            zero = jnp.zeros((LANES,), jnp.float32)
            acc_s = [zero, zero, zero, zero]
            acc_q = [zero, zero, zero, zero]
            for j in range(NVEC):
                sl = pl.ds(j * LANES, LANES)
                v = x_v[r, sl] + g_v[r, sl]
                x_v[r, sl] = v
                acc_s[j % 4] = acc_s[j % 4] + v
                acc_q[j % 4] = acc_q[j % 4] + v * v
            s = (acc_s[0] + acc_s[1]) + (acc_s[2] + acc_s[3])
            q = (acc_q[0] + acc_q[1]) + (acc_q[2] + acc_q[3])
            lane = lax.iota(jnp.int32, LANES)
            for k in (8, 4, 2, 1):
                red_v[pl.ds(0, LANES)] = s
                red_v[pl.ds(LANES, LANES)] = q
                perm = lax.bitwise_xor(lane, k)
                s = s + plsc.load_gather(red_v, [perm])
                q = q + plsc.load_gather(red_v, [perm + LANES])
            m = s * (1.0 / D_MODEL)
            var = q * (1.0 / D_MODEL) - m * m
            vv = var + LN_EPS
            bits = plsc.bitcast(vv, jnp.int32)
            y = plsc.bitcast(
                jnp.full((LANES,), 0x5F3759DF, jnp.int32)
                - lax.shift_right_arithmetic(bits, 1),
                jnp.float32,
            )
            h = vv * 0.5
            for _ in range(4):
                y = y * (1.5 - h * y * y)
            for j in range(NVEC):
                sl = pl.ds(j * LANES, LANES)
                v = x_v[r, sl]
                x_v[r, sl] = (v - m) * y * gb_v[0, sl] + gb_v[1, sl]
            return rcarry

        lax.fori_loop(0, CH, row_body, 0)
        pltpu.sync_copy(x_v, out_hbm.at[pl.ds(off, CH)])
        return carry

    lax.fori_loop(0, NCHUNK, chunk_body, 0)


def _fused_body(x_ref, pid_ref, tid_ref, t_ref, g_ref, b_ref, o_ref):
    x = x_ref[...]
    pid = pid_ref[...]
    tid = tid_ref[...]
    iota = jax.lax.broadcasted_iota(jnp.int32, (BLK, TAB), 1)
    sel = ((iota == pid) | (iota == tid + TOK_OFF)).astype(jnp.float32)
    x = x + jnp.dot(sel, t_ref[...], preferred_element_type=jnp.float32)
    mean = jnp.mean(x, axis=1, keepdims=True)
    xc = x - mean
    var = jnp.mean(xc * xc, axis=1, keepdims=True)
    o_ref[...] = xc * (jax.lax.rsqrt(var + LN_EPS) * g_ref[...]) + b_ref[...]


@jax.jit
def kernel(token_type_ids, position_ids, inputs_embeds, W_pos, b_pos,
           W_tok, b_tok, ln_gamma, ln_beta):
    B, S, D = inputs_embeds.shape
    n = B * S
    x = inputs_embeds.reshape(n, D)
    pid = position_ids.reshape(n).astype(jnp.int32)
    tid = token_type_ids.reshape(n).astype(jnp.int32)

    ctab, ttab = pl.pallas_call(
        _proj_body,
        out_shape=(
            jax.ShapeDtypeStruct((CTAB, D_MODEL), jnp.float32),
            jax.ShapeDtypeStruct((TAB, D_MODEL), jnp.float32),
        ),
    )(
        jnp.asarray(_PE_POS), jnp.asarray(_PE_TOK),
        W_pos.T, b_pos.reshape(1, D_MODEL),
        W_tok.T, b_tok.reshape(1, D_MODEL),
        jnp.asarray(_R_EXP), jnp.asarray(_S_EXP),
    )

    n_tc = n - N_SC
    sc_blocks = N_SC // BLK
    out_tc = pl.pallas_call(
        _fused_body,
        grid=(n_tc // BLK,),
        in_specs=[
            pl.BlockSpec((BLK, D_MODEL), lambda i: (i + sc_blocks, 0)),
            pl.BlockSpec((BLK, 1), lambda i: (i + sc_blocks, 0)),
            pl.BlockSpec((BLK, 1), lambda i: (i + sc_blocks, 0)),
            pl.BlockSpec((TAB, D_MODEL), lambda i: (0, 0)),
            pl.BlockSpec((1, D_MODEL), lambda i: (0, 0)),
            pl.BlockSpec((1, D_MODEL), lambda i: (0, 0)),
        ],
        out_specs=pl.BlockSpec((BLK, D_MODEL), lambda i: (i, 0)),
        out_shape=jax.ShapeDtypeStruct((n_tc, D_MODEL), jnp.float32),
    )(
        x, pid.reshape(n, 1), tid.reshape(n, 1), ttab,
        ln_gamma.reshape(1, D_MODEL), ln_beta.reshape(1, D_MODEL),
    )

    mesh = plsc.VectorSubcoreMesh(
        core_axis_name="c", subcore_axis_name="s", num_cores=NC)
    sc = functools.partial(
        pl.kernel,
        mesh=mesh,
        compiler_params=pltpu.CompilerParams(needs_layout_passes=False),
        out_type=jax.ShapeDtypeStruct((N_SC, D), jnp.float32),
        scratch_types=[
            pltpu.VMEM((CH, D_MODEL), jnp.float32),
            pltpu.VMEM((CH, D_MODEL), jnp.float32),
            pltpu.VMEM((CH,), jnp.int32),
            pltpu.VMEM((CH,), jnp.int32),
            pltpu.VMEM((CH,), jnp.int32),
            pltpu.VMEM((2, D_MODEL), jnp.float32),
            pltpu.VMEM((2 * LANES,), jnp.float32),
            pltpu.SemaphoreType.DMA,
        ],
    )(_sc_body)
    out_sc = sc(x, pid, tid, ctab, ln_gamma, ln_beta)

    out = jnp.concatenate([out_sc, out_tc], axis=0)
    return out.reshape(B, S, D)
